# baseline (speedup 1.0000x reference)
import functools

import jax
import jax.numpy as jnp
from jax import lax
from jax.experimental import pallas as pl
from jax.experimental.pallas import tpu as pltpu
from jax.experimental.pallas import tpu_sc as plsc

NUM_EMB = 1000000
D = 64
B = 4096
S = 200
NC = 2
NS = 16
NW = NC * NS
ROWS_W = B // NW
SA = 104
SB = S - SA
NBUF = 4
N_ROUND = ROWS_W // NBUF

_mesh = plsc.VectorSubcoreMesh(core_axis_name="c", subcore_axis_name="s")


@functools.partial(
    pl.kernel,
    out_type=jax.ShapeDtypeStruct((B, S, D), jnp.float32),
    mesh=_mesh,
    compiler_params=pltpu.CompilerParams(use_tc_tiling_on_sc=False),
    scratch_types=[
        pltpu.VMEM((ROWS_W, SA), jnp.int32),
        pltpu.VMEM((ROWS_W, SB), jnp.int32),
        pltpu.VMEM((NBUF, S, D), jnp.float32),
        pltpu.SemaphoreType.DMA((NBUF,)),
        pltpu.SemaphoreType.DMA((NBUF,)),
    ],
)
def _embed_sc(table_hbm, idx_hbm, out_hbm, idx_a, idx_b, rows_v, gsem, ssem):
    wid = lax.axis_index("s") * NC + lax.axis_index("c")
    r0 = wid * ROWS_W
    pltpu.sync_copy(idx_hbm.at[pl.ds(r0, ROWS_W), pl.ds(0, SA)], idx_a)
    pltpu.sync_copy(idx_hbm.at[pl.ds(r0, ROWS_W), pl.ds(SA, SB)], idx_b)

    def fire_gather(g, b):
        pltpu.async_copy(
            table_hbm.at[idx_a.at[g]], rows_v.at[b, pl.ds(0, SA)], gsem.at[b])
        pltpu.async_copy(
            table_hbm.at[idx_b.at[g]], rows_v.at[b, pl.ds(SA, SB)], gsem.at[b])

    def wait_gather(b):
        pltpu.make_async_copy(
            table_hbm.at[idx_a.at[0]], rows_v.at[b], gsem.at[b]).wait()

    def fire_store(g, b):
        pltpu.async_copy(rows_v.at[b], out_hbm.at[r0 + g], ssem.at[b])

    def wait_store(b):
        pltpu.make_async_copy(rows_v.at[b], out_hbm.at[r0], ssem.at[b]).wait()

    for b in range(NBUF):
        fire_gather(b, b)

    @pl.loop(0, N_ROUND - 1)
    def _round(r):
        g0 = r * NBUF
        for b in range(NBUF):
            wait_gather(b)
            fire_store(g0 + b, b)
        for b in range(NBUF):
            wait_store(b)
            fire_gather(g0 + NBUF + b, b)

    g0 = (N_ROUND - 1) * NBUF
    for b in range(NBUF):
        wait_gather(b)
        fire_store(g0 + b, b)
    for b in range(NBUF):
        wait_store(b)


def kernel(token_ids, embeddings):
    return _embed_sc(embeddings, token_ids)


# speedup vs baseline: 1.2241x; 1.2241x over previous
import functools

import jax
import jax.numpy as jnp
from jax import lax
from jax.experimental import pallas as pl
from jax.experimental.pallas import tpu as pltpu
from jax.experimental.pallas import tpu_sc as plsc

NUM_EMB = 1000000
D = 64
DP = 128
B = 4096
S = 200
TOTAL = B * S
NC = 2
NS = 16
NW = NC * NS
PER_W = TOTAL // NW
CHUNK = 128
N_CHUNK = PER_W // CHUNK
NBUF = 4
N_ROUND = N_CHUNK // NBUF

_mesh = plsc.VectorSubcoreMesh(core_axis_name="c", subcore_axis_name="s")


@functools.partial(
    pl.kernel,
    out_type=jax.ShapeDtypeStruct((TOTAL, DP), jnp.float32),
    mesh=_mesh,
    compiler_params=pltpu.CompilerParams(use_tc_tiling_on_sc=False),
    scratch_types=[
        pltpu.VMEM((N_CHUNK, CHUNK), jnp.int32),
        pltpu.VMEM((NBUF, CHUNK, DP), jnp.float32),
        pltpu.SemaphoreType.DMA((NBUF,)),
        pltpu.SemaphoreType.DMA((NBUF,)),
    ],
)
def _embed_sc(table_hbm, idx_hbm, out_hbm, idx_v, rows_v, gsem, ssem):
    wid = lax.axis_index("s") * NC + lax.axis_index("c")
    pltpu.sync_copy(idx_hbm.at[pl.ds(wid * N_CHUNK, N_CHUNK)], idx_v)
    base = wid * PER_W

    def fire_gather(g, b):
        pltpu.async_copy(table_hbm.at[idx_v.at[g]], rows_v.at[b], gsem.at[b])

    def wait_gather(b):
        pltpu.make_async_copy(
            table_hbm.at[idx_v.at[0]], rows_v.at[b], gsem.at[b]).wait()

    def fire_store(g, b):
        pltpu.async_copy(
            rows_v.at[b], out_hbm.at[pl.ds(base + g * CHUNK, CHUNK)],
            ssem.at[b])

    def wait_store(b):
        pltpu.make_async_copy(
            rows_v.at[b], out_hbm.at[pl.ds(base, CHUNK)], ssem.at[b]).wait()

    for b in range(NBUF):
        fire_gather(b, b)

    @pl.loop(0, N_ROUND - 1)
    def _round(r):
        g0 = r * NBUF
        for b in range(NBUF):
            wait_gather(b)
            fire_store(g0 + b, b)
        for b in range(NBUF):
            wait_store(b)
            fire_gather(g0 + NBUF + b, b)

    g0 = (N_ROUND - 1) * NBUF
    for b in range(NBUF):
        wait_gather(b)
        fire_store(g0 + b, b)
    for b in range(NBUF):
        wait_store(b)


def kernel(token_ids, embeddings):
    padded = jnp.pad(embeddings, ((0, 0), (0, DP - D)))
    flat = token_ids.reshape(NW * N_CHUNK, CHUNK)
    res = _embed_sc(padded, flat)
    return res[:, :D].reshape(B, S, D)
